# baseline (speedup 1.0000x reference)
import jax
import jax.numpy as jnp
from jax import lax
from jax.experimental import pallas as pl
from jax.experimental.pallas import tpu as pltpu
from jax.experimental.pallas import tpu_sc as plsc

_NC = 2
_NS = 16
_NW = _NC * _NS

_B = 16384
_P = 20
_N = 50
_NJ = _P + _N
_D = 64
_CB = 4
_VCH = _CB * _NJ
_POS0 = 0
_NEG0 = _CB * _P
_BW = _B // _NW
_NCHUNK = _BW // _CB
_UIDX_W = 128
_NUIDX = _BW // _UIDX_W


def _sc_body(pu_hbm, posv_hbm, negv_hbm, w_hbm, out_hbm,
             pu_v, uidx_v, pidx, nidx, ubuf, vbuf0, vbuf1, sbuf0, sbuf1,
             usem, gsem0, gsem1, ssem0, ssem1):
    cid = lax.axis_index("c")
    sid = lax.axis_index("s")
    wid = sid * _NC + cid
    lane = lax.iota(jnp.int32, 16)
    last = lane == 15

    def fetch(c, vbuf, gsem):
        for r in range(_CB):
            pltpu.async_copy(
                w_hbm.at[pidx.at[c * _CB + r]],
                vbuf.at[pl.ds(_POS0 + r * _P, _P)], gsem)
        for r in range(_CB):
            pltpu.async_copy(
                w_hbm.at[nidx.at[c * _CB + r]],
                vbuf.at[pl.ds(_NEG0 + r * _N, _N)], gsem)

    def drain_gather(vbuf, gsem):
        pltpu.make_async_copy(
            w_hbm.at[pl.ds(0, _VCH)], vbuf, gsem).wait()

    def drain_scores(sbuf, ssem):
        pltpu.make_async_copy(
            sbuf, out_hbm.at[pl.ds(0, _VCH)], ssem).wait()

    def compute(c, vbuf, sbuf):
        def b_body(b, carry2):
            lb = c * _CB + b
            u0, u1 = plsc.unpack(ubuf[lb, pl.ds(0, 32)],
                                 format=plsc.PackFormat.INTERLEAVED)
            u2, u3 = plsc.unpack(ubuf[lb, pl.ds(32, 32)],
                                 format=plsc.PackFormat.INTERLEAVED)
            us = [u0, u1, u2, u3]

            def dot_store(vrow, p):
                v0, v1 = plsc.unpack(vbuf[vrow, pl.ds(0, 32)],
                                     format=plsc.PackFormat.INTERLEAVED)
                v2, v3 = plsc.unpack(vbuf[vrow, pl.ds(32, 32)],
                                     format=plsc.PackFormat.INTERLEAVED)
                t = us[0] * v0
                t = t + us[1] * v1
                t = t + us[2] * v2
                t = t + us[3] * v3
                cs = plsc.cumsum(t)
                idx = jnp.broadcast_to(p, (16,)).astype(jnp.int32)
                plsc.store_scatter(sbuf, [idx], cs, mask=last)

            @plsc.parallel_loop(0, _P, unroll=5)
            def _(j):
                dot_store(_POS0 + b * _P + j, b * _NJ + j)

            @plsc.parallel_loop(0, _N, unroll=5)
            def _(j):
                dot_store(_NEG0 + b * _N + j, b * _NJ + _P + j)

            return carry2

        lax.fori_loop(0, _CB, b_body, 0)

    row0 = wid * _BW
    pltpu.async_copy(pu_hbm.at[pl.ds(row0, _BW)], pu_v, usem)
    pltpu.async_copy(posv_hbm.at[pl.ds(row0, _BW)], pidx, usem)
    pltpu.async_copy(negv_hbm.at[pl.ds(row0, _BW)], nidx, usem)
    pltpu.make_async_copy(pu_hbm.at[pl.ds(0, _BW)], pu_v, usem).wait()
    zero16 = jnp.broadcast_to(jnp.int32(0), (16,))
    for k in range(_BW // 16):
        vals = plsc.load_gather(pu_v, [k * 16 + lane, zero16])
        uidx_v[k * 16 // _UIDX_W,
               pl.ds((k * 16) % _UIDX_W, 16)] = vals
    for k in range(_NUIDX):
        pltpu.async_copy(
            w_hbm.at[uidx_v.at[k]],
            ubuf.at[pl.ds(k * _UIDX_W, _UIDX_W)], usem)
    pltpu.make_async_copy(posv_hbm.at[pl.ds(0, _BW)], pidx, usem).wait()
    pltpu.make_async_copy(negv_hbm.at[pl.ds(0, _BW)], nidx, usem).wait()
    pltpu.make_async_copy(w_hbm.at[pl.ds(0, _BW)], ubuf, usem).wait()
    fetch(0, vbuf0, gsem0)
    g0 = wid * _NCHUNK

    def pair_body(i, carry):
        ca = 2 * i
        cb = 2 * i + 1
        fetch(cb, vbuf1, gsem1)
        drain_gather(vbuf0, gsem0)

        @pl.when(i >= 1)
        def _():
            drain_scores(sbuf0, ssem0)

        compute(ca, vbuf0, sbuf0)
        pltpu.async_copy(sbuf0, out_hbm.at[pl.ds((g0 + ca) * _VCH, _VCH)],
                         ssem0)

        @pl.when(i < _NCHUNK // 2 - 1)
        def _():
            fetch(cb + 1, vbuf0, gsem0)

        drain_gather(vbuf1, gsem1)

        @pl.when(i >= 1)
        def _():
            drain_scores(sbuf1, ssem1)

        compute(cb, vbuf1, sbuf1)
        pltpu.async_copy(sbuf1, out_hbm.at[pl.ds((g0 + cb) * _VCH, _VCH)],
                         ssem1)
        return carry

    lax.fori_loop(0, _NCHUNK // 2, pair_body, 0)
    drain_scores(sbuf0, ssem0)
    drain_scores(sbuf1, ssem1)


def _sc_scores(pos_u, pos_v, neg_v, W):
    mesh = plsc.VectorSubcoreMesh(
        core_axis_name="c", subcore_axis_name="s",
        num_cores=_NC, num_subcores=_NS)
    return pl.kernel(
        _sc_body,
        out_type=jax.ShapeDtypeStruct((_B * _NJ,), jnp.float32),
        mesh=mesh,
        compiler_params=pltpu.CompilerParams(
            needs_layout_passes=False, use_tc_tiling_on_sc=False),
        scratch_types=[
            pltpu.VMEM((_BW, 1), jnp.int32),
            pltpu.VMEM((_NUIDX, _UIDX_W), jnp.int32),
            pltpu.VMEM((_BW, _P), jnp.int32),
            pltpu.VMEM((_BW, _N), jnp.int32),
            pltpu.VMEM((_BW, _D), jnp.bfloat16),
            pltpu.VMEM((_VCH, _D), jnp.bfloat16),
            pltpu.VMEM((_VCH, _D), jnp.bfloat16),
            pltpu.VMEM((_VCH,), jnp.float32),
            pltpu.VMEM((_VCH,), jnp.float32),
            pltpu.SemaphoreType.DMA,
            pltpu.SemaphoreType.DMA,
            pltpu.SemaphoreType.DMA,
            pltpu.SemaphoreType.DMA,
            pltpu.SemaphoreType.DMA,
        ],
    )(pos_u, pos_v, neg_v, W)


_BR = 1120


def _tc_loss_body(x_ref, o_ref):
    i = pl.program_id(0)
    x = x_ref[...]
    rowi = lax.broadcasted_iota(jnp.int32, (_BR, 128), 0)
    coli = lax.broadcasted_iota(jnp.int32, (_BR, 128), 1)
    p = (i * _BR + rowi) * 128 + coli
    j = p % _NJ
    s = jnp.where(j < _P, x, -x)
    ls = jnp.minimum(s, 0.0) - jnp.log1p(jnp.exp(-jnp.abs(s)))
    part = jnp.sum(ls)

    @pl.when(i == 0)
    def _():
        o_ref[...] = jnp.zeros((1, 1), jnp.float32)

    o_ref[...] += part

    @pl.when(i == pl.num_programs(0) - 1)
    def _():
        o_ref[...] = -o_ref[...] / float(_B * _NJ)


def _tc_loss(scores2d):
    nrow = scores2d.shape[0]
    return pl.pallas_call(
        _tc_loss_body,
        grid=(nrow // _BR,),
        in_specs=[pl.BlockSpec((_BR, 128), lambda i: (i, 0))],
        out_specs=pl.BlockSpec((1, 1), lambda i: (0, 0)),
        out_shape=jax.ShapeDtypeStruct((1, 1), jnp.float32),
    )(scores2d)


def kernel(pos_u, pos_v, neg_v, W):
    W_lin = W.astype(jnp.bfloat16).reshape(64000000).reshape(1000000, 64)
    scores = _sc_scores(pos_u.astype(jnp.int32), pos_v.astype(jnp.int32),
                        neg_v.astype(jnp.int32), W_lin)
    loss = _tc_loss(scores.reshape(_B * _NJ // 128, 128))
    return loss[0, 0]


# speedup vs baseline: 1.2649x; 1.2649x over previous
import jax
import jax.numpy as jnp
from jax import lax
from jax.experimental import pallas as pl
from jax.experimental.pallas import tpu as pltpu
from jax.experimental.pallas import tpu_sc as plsc

_NC = 2
_NS = 16
_NW = _NC * _NS

_B = 16384
_P = 20
_N = 50
_NJ = _P + _N
_D = 64
_CB = 4
_VCH = _CB * _NJ
_POS0 = 0
_NEG0 = _CB * _P
_BW = _B // _NW
_NCHUNK = _BW // _CB
_UIDX_W = 128
_NUIDX = _BW // _UIDX_W


def _sc_body(uidx_hbm, posv_hbm, negv_hbm, w_hbm, out_hbm,
             uidx_v, pidx, nidx, ubuf, vbuf0, vbuf1, sbuf0, sbuf1,
             usem, gsem0, gsem1, ssem0, ssem1):
    cid = lax.axis_index("c")
    sid = lax.axis_index("s")
    wid = sid * _NC + cid
    lane = lax.iota(jnp.int32, 16)
    last = lane == 15

    def fetch(c, vbuf, gsem):
        for r in range(_CB):
            pltpu.async_copy(
                w_hbm.at[pidx.at[c * _CB + r]],
                vbuf.at[pl.ds(_POS0 + r * _P, _P)], gsem)
        for r in range(_CB):
            pltpu.async_copy(
                w_hbm.at[nidx.at[c * _CB + r]],
                vbuf.at[pl.ds(_NEG0 + r * _N, _N)], gsem)

    def drain_gather(vbuf, gsem):
        pltpu.make_async_copy(
            w_hbm.at[pl.ds(0, _VCH)], vbuf, gsem).wait()

    def drain_scores(sbuf, ssem):
        pltpu.make_async_copy(
            sbuf, out_hbm.at[pl.ds(0, _VCH)], ssem).wait()

    def compute(c, vbuf, sbuf):
        def b_body(b, carry2):
            lb = c * _CB + b
            us = [ubuf[lb, pl.ds(16 * k, 16)] for k in range(4)]

            def dot_store(vrow, p):
                t = us[0] * vbuf[vrow, pl.ds(0, 16)]
                t = t + us[1] * vbuf[vrow, pl.ds(16, 16)]
                t = t + us[2] * vbuf[vrow, pl.ds(32, 16)]
                t = t + us[3] * vbuf[vrow, pl.ds(48, 16)]
                cs = plsc.cumsum(t)
                idx = jnp.broadcast_to(p, (16,)).astype(jnp.int32)
                plsc.store_scatter(sbuf, [idx], cs, mask=last)

            @plsc.parallel_loop(0, _P, unroll=5)
            def _(j):
                dot_store(_POS0 + b * _P + j, b * _NJ + j)

            @plsc.parallel_loop(0, _N, unroll=5)
            def _(j):
                dot_store(_NEG0 + b * _N + j, b * _NJ + _P + j)

            return carry2

        lax.fori_loop(0, _CB, b_body, 0)

    row0 = wid * _BW
    pltpu.sync_copy(uidx_hbm.at[pl.ds(wid * _NUIDX, _NUIDX)], uidx_v)
    pltpu.async_copy(posv_hbm.at[pl.ds(row0, _BW)], pidx, usem)
    pltpu.async_copy(negv_hbm.at[pl.ds(row0, _BW)], nidx, usem)
    for k in range(_NUIDX):
        pltpu.async_copy(
            w_hbm.at[uidx_v.at[k]],
            ubuf.at[pl.ds(k * _UIDX_W, _UIDX_W)], usem)
    pltpu.make_async_copy(posv_hbm.at[pl.ds(0, _BW)], pidx, usem).wait()
    pltpu.make_async_copy(negv_hbm.at[pl.ds(0, _BW)], nidx, usem).wait()
    pltpu.make_async_copy(w_hbm.at[pl.ds(0, _BW)], ubuf, usem).wait()
    fetch(0, vbuf0, gsem0)
    g0 = wid * _NCHUNK

    def pair_body(i, carry):
        ca = 2 * i
        cb = 2 * i + 1
        fetch(cb, vbuf1, gsem1)
        drain_gather(vbuf0, gsem0)

        @pl.when(i >= 1)
        def _():
            drain_scores(sbuf0, ssem0)

        compute(ca, vbuf0, sbuf0)
        pltpu.async_copy(sbuf0, out_hbm.at[pl.ds((g0 + ca) * _VCH, _VCH)],
                         ssem0)

        @pl.when(i < _NCHUNK // 2 - 1)
        def _():
            fetch(cb + 1, vbuf0, gsem0)

        drain_gather(vbuf1, gsem1)

        @pl.when(i >= 1)
        def _():
            drain_scores(sbuf1, ssem1)

        compute(cb, vbuf1, sbuf1)
        pltpu.async_copy(sbuf1, out_hbm.at[pl.ds((g0 + cb) * _VCH, _VCH)],
                         ssem1)
        return carry

    lax.fori_loop(0, _NCHUNK // 2, pair_body, 0)
    drain_scores(sbuf0, ssem0)
    drain_scores(sbuf1, ssem1)


def _sc_scores(u_idx, pos_v, neg_v, W):
    mesh = plsc.VectorSubcoreMesh(
        core_axis_name="c", subcore_axis_name="s",
        num_cores=_NC, num_subcores=_NS)
    return pl.kernel(
        _sc_body,
        out_type=jax.ShapeDtypeStruct((_B * _NJ,), jnp.float32),
        mesh=mesh,
        compiler_params=pltpu.CompilerParams(
            needs_layout_passes=False, use_tc_tiling_on_sc=False),
        scratch_types=[
            pltpu.VMEM((_NUIDX, _UIDX_W), jnp.int32),
            pltpu.VMEM((_BW, _P), jnp.int32),
            pltpu.VMEM((_BW, _N), jnp.int32),
            pltpu.VMEM((_BW, _D), jnp.float32),
            pltpu.VMEM((_VCH, _D), jnp.float32),
            pltpu.VMEM((_VCH, _D), jnp.float32),
            pltpu.VMEM((_VCH,), jnp.float32),
            pltpu.VMEM((_VCH,), jnp.float32),
            pltpu.SemaphoreType.DMA,
            pltpu.SemaphoreType.DMA,
            pltpu.SemaphoreType.DMA,
            pltpu.SemaphoreType.DMA,
            pltpu.SemaphoreType.DMA,
        ],
    )(u_idx, pos_v, neg_v, W)


_BR = 1120


def _tc_loss_body(x_ref, o_ref):
    i = pl.program_id(0)
    x = x_ref[...]
    rowi = lax.broadcasted_iota(jnp.int32, (_BR, 128), 0)
    coli = lax.broadcasted_iota(jnp.int32, (_BR, 128), 1)
    p = (i * _BR + rowi) * 128 + coli
    j = p % _NJ
    s = jnp.where(j < _P, x, -x)
    ls = jnp.minimum(s, 0.0) - jnp.log1p(jnp.exp(-jnp.abs(s)))
    part = jnp.sum(ls)

    @pl.when(i == 0)
    def _():
        o_ref[...] = jnp.zeros((1, 1), jnp.float32)

    o_ref[...] += part

    @pl.when(i == pl.num_programs(0) - 1)
    def _():
        o_ref[...] = -o_ref[...] / float(_B * _NJ)


def _tc_loss(scores2d):
    nrow = scores2d.shape[0]
    return pl.pallas_call(
        _tc_loss_body,
        grid=(nrow // _BR,),
        in_specs=[pl.BlockSpec((_BR, 128), lambda i: (i, 0))],
        out_specs=pl.BlockSpec((1, 1), lambda i: (0, 0)),
        out_shape=jax.ShapeDtypeStruct((1, 1), jnp.float32),
    )(scores2d)


def kernel(pos_u, pos_v, neg_v, W):
    u_idx = pos_u.reshape(_B // _UIDX_W, _UIDX_W).astype(jnp.int32)
    scores = _sc_scores(u_idx, pos_v.astype(jnp.int32),
                        neg_v.astype(jnp.int32), W)
    loss = _tc_loss(scores.reshape(_B * _NJ // 128, 128))
    return loss[0, 0]
